# Initial kernel scaffold; baseline (speedup 1.0000x reference)
import jax
import jax.numpy as jnp
from jax.experimental import pallas as pl


def kernel(x, conv_w, conv_b, bn_gamma, bn_beta):
    raise NotImplementedError("write your pallas kernel here")


# speedup vs baseline: 5.7968x; 5.7968x over previous
import functools

import jax
import jax.numpy as jnp
from jax import lax
from jax.experimental import pallas as pl
from jax.experimental.pallas import tpu as pltpu

EPS = 1e-5
NEG_SLOPE = 0.2
LANES = 128


def _conv_acc(xp_ref, w_ref, h_out, w_out, kh, kw):
    c_in = xp_ref.shape[-1]
    acc = None
    for dy in range(kh):
        for dx in range(kw):
            a = xp_ref[0, dy:dy + h_out, dx:dx + w_out, :]
            a = a.reshape(h_out * w_out, c_in)
            k0 = (dy * kw + dx) * c_in
            p = jnp.dot(a, w_ref[k0:k0 + c_in, :],
                        preferred_element_type=jnp.float32)
            acc = p if acc is None else acc + p
    return acc


def _stats_kernel(xp_ref, w_ref, sum_ref, sumsq_ref, *, h_out, w_out, kh, kw):
    y = _conv_acc(xp_ref, w_ref, h_out, w_out, kh, kw)
    sum_ref[0, 0, :] = jnp.sum(y, axis=0)
    sumsq_ref[0, 0, :] = jnp.sum(y * y, axis=0)


def _bn_lrelu_kernel(xp_ref, w_ref, scale_ref, shift_ref, out_ref,
                     *, h_out, w_out, kh, kw):
    y = _conv_acc(xp_ref, w_ref, h_out, w_out, kh, kw)
    z = y * scale_ref[...] + shift_ref[...]
    z = jnp.where(z >= 0, z, NEG_SLOPE * z)
    out_ref[0] = z.T


def kernel(x, conv_w, conv_b, bn_gamma, bn_beta):
    del conv_b
    pad = 1
    n, c_in, h, w = x.shape
    c_out, _, kh, kw = conv_w.shape
    h_out = h + 2 * pad - kh + 1
    w_out = w + 2 * pad - kw + 1
    m = n * h_out * w_out
    hwo = h_out * w_out

    xp = jnp.pad(jnp.transpose(x, (0, 2, 3, 1)),
                 ((0, 0), (pad, pad), (pad, pad), (0, 0)))
    xp = xp.astype(jnp.bfloat16)
    wt = jnp.transpose(conv_w, (2, 3, 1, 0)).reshape(kh * kw * c_in, c_out)
    wt = jnp.pad(wt, ((0, 0), (0, LANES - c_out))).astype(jnp.bfloat16)

    cparams = pltpu.CompilerParams(
        dimension_semantics=("parallel",),
        vmem_limit_bytes=48 * 1024 * 1024,
    )
    xp_spec = pl.BlockSpec((1, h + 2 * pad, w + 2 * pad, c_in),
                           lambda i: (i, 0, 0, 0))
    w_spec = pl.BlockSpec((kh * kw * c_in, LANES), lambda i: (0, 0))
    conv_flops = 2 * hwo * kh * kw * c_in * LANES

    sums, sumsqs = pl.pallas_call(
        functools.partial(_stats_kernel, h_out=h_out, w_out=w_out, kh=kh, kw=kw),
        out_shape=(jax.ShapeDtypeStruct((n, 1, LANES), jnp.float32),
                   jax.ShapeDtypeStruct((n, 1, LANES), jnp.float32)),
        grid=(n,),
        in_specs=[xp_spec, w_spec],
        out_specs=(pl.BlockSpec((1, 1, LANES), lambda i: (i, 0, 0)),
                   pl.BlockSpec((1, 1, LANES), lambda i: (i, 0, 0))),
        compiler_params=cparams,
        cost_estimate=pl.CostEstimate(
            flops=n * conv_flops, transcendentals=0,
            bytes_accessed=xp.size * 2 + wt.size * 2 + 2 * n * LANES * 4),
    )(xp, wt)

    mean = jnp.sum(sums[:, 0, :], axis=0) / m
    ex2 = jnp.sum(sumsqs[:, 0, :], axis=0) / m
    var = jnp.maximum(ex2 - mean * mean, 0.0)
    inv_std = lax.rsqrt(var + EPS)
    gamma_pad = jnp.pad(bn_gamma.astype(jnp.float32), (0, LANES - c_out))
    beta_pad = jnp.pad(bn_beta.astype(jnp.float32), (0, LANES - c_out))
    scale = (gamma_pad * inv_std).reshape(1, LANES)
    shift = (beta_pad - mean * gamma_pad * inv_std).reshape(1, LANES)

    out_t = pl.pallas_call(
        functools.partial(_bn_lrelu_kernel, h_out=h_out, w_out=w_out,
                          kh=kh, kw=kw),
        out_shape=jax.ShapeDtypeStruct((n, LANES, hwo), jnp.float32),
        grid=(n,),
        in_specs=[xp_spec, w_spec,
                  pl.BlockSpec((1, LANES), lambda i: (0, 0)),
                  pl.BlockSpec((1, LANES), lambda i: (0, 0))],
        out_specs=pl.BlockSpec((1, LANES, hwo), lambda i: (i, 0, 0)),
        compiler_params=cparams,
        cost_estimate=pl.CostEstimate(
            flops=n * conv_flops + 4 * m * LANES, transcendentals=0,
            bytes_accessed=xp.size * 2 + wt.size * 2 + m * LANES * 4),
    )(xp, wt, scale, shift)

    return out_t.reshape(n, LANES, h_out, w_out)[:, :c_out]
